# Initial kernel scaffold; baseline (speedup 1.0000x reference)
import jax
import jax.numpy as jnp
from jax.experimental import pallas as pl


def kernel(x_, conv1_w, bn1_gamma, bn1_beta, bn1_mean, bn1_var, conv2_w, conv2_b, bn2_gamma, bn2_beta, bn2_mean, bn2_var, centroids, length):
    raise NotImplementedError("write your pallas kernel here")


# speedup vs baseline: 1.8306x; 1.8306x over previous
import jax
import jax.numpy as jnp
from jax.experimental import pallas as pl
from jax.experimental.pallas import tpu as pltpu

EPS_BN = 1e-5
EPS_NORM = 1e-12
NEG_FILL = -1e18


def _netvlad_kernel(length_ref, x_ref, taps_ref, shift1_ref, w2_ref,
                    bias2_ref, cent_ref, out_ref):
    n = pl.program_id(0)
    T, C = x_ref.shape[1], x_ref.shape[2]
    K = cent_ref.shape[0]

    x = x_ref[0]
    ss = jnp.sum(x * x, axis=1, keepdims=True)
    inv = 1.0 / jnp.maximum(jnp.sqrt(ss), EPS_NORM)
    xn = x * inv

    zrow = jnp.zeros((1, C), jnp.float32)
    prev = jnp.concatenate([zrow, xn[:-1, :]], axis=0)
    nxt = jnp.concatenate([xn[1:, :], zrow], axis=0)
    h = (prev * taps_ref[0:1, :] + xn * taps_ref[1:2, :]
         + nxt * taps_ref[2:3, :] + shift1_ref[0:1, :])
    h = jnp.maximum(h, 0.0)

    s = jnp.dot(h, w2_ref[...], preferred_element_type=jnp.float32)
    s = jnp.maximum(s + bias2_ref[0:1, :], 0.0)

    t_idx = jax.lax.broadcasted_iota(jnp.int32, (T, 1), 0)
    valid = t_idx < length_ref[n]
    s = jnp.where(valid, s, NEG_FILL)
    m = jnp.max(s, axis=1, keepdims=True)
    e = jnp.exp(s - m)
    a = e * (1.0 / jnp.sum(e, axis=1, keepdims=True))

    vlad = jax.lax.dot_general(a, xn, (((0,), (0,)), ((), ())),
                               preferred_element_type=jnp.float32)
    ones = jnp.ones((T, 1), jnp.float32)
    asum = jax.lax.dot_general(a, ones, (((0,), (0,)), ((), ())),
                               preferred_element_type=jnp.float32)
    vlad = vlad - asum * cent_ref[...]

    n2 = jnp.sum(vlad * vlad, axis=1, keepdims=True)
    vlad = vlad * (1.0 / jnp.maximum(jnp.sqrt(n2), EPS_NORM))
    g = jnp.sum(vlad * vlad)
    vlad = vlad * (1.0 / jnp.maximum(jnp.sqrt(g), EPS_NORM))
    out_ref[0] = vlad


def kernel(x_, conv1_w, bn1_gamma, bn1_beta, bn1_mean, bn1_var,
           conv2_w, conv2_b, bn2_gamma, bn2_beta, bn2_mean, bn2_var,
           centroids, length):
    N, T, C = x_.shape
    K = centroids.shape[0]

    scale1 = bn1_gamma * jax.lax.rsqrt(bn1_var + EPS_BN)
    shift1 = (bn1_beta - bn1_mean * scale1).reshape(1, C)
    taps = conv1_w[:, 0, :, 1].T * scale1[None, :]

    scale2 = bn2_gamma * jax.lax.rsqrt(bn2_var + EPS_BN)
    w2 = conv2_w[:, :, 0, 0].T * scale2[None, :]
    bias2 = (conv2_b * scale2 + bn2_beta - bn2_mean * scale2).reshape(1, K)

    out = pl.pallas_call(
        _netvlad_kernel,
        grid=(N,),
        in_specs=[
            pl.BlockSpec(memory_space=pltpu.SMEM),
            pl.BlockSpec((1, T, C), lambda n: (n, 0, 0)),
            pl.BlockSpec((3, C), lambda n: (0, 0)),
            pl.BlockSpec((1, C), lambda n: (0, 0)),
            pl.BlockSpec((C, K), lambda n: (0, 0)),
            pl.BlockSpec((1, K), lambda n: (0, 0)),
            pl.BlockSpec((K, C), lambda n: (0, 0)),
        ],
        out_specs=pl.BlockSpec((1, K, C), lambda n: (n, 0, 0)),
        out_shape=jax.ShapeDtypeStruct((N, K, C), jnp.float32),
        compiler_params=pltpu.CompilerParams(
            dimension_semantics=("parallel",),
        ),
    )(length, x_, taps, shift1, w2, bias2, centroids)
    return out.reshape(N, K * C)


# speedup vs baseline: 2.3773x; 1.2986x over previous
import jax
import jax.numpy as jnp
from jax.experimental import pallas as pl
from jax.experimental.pallas import tpu as pltpu

EPS_BN = 1e-5
EPS_NORM = 1e-12
NEG_FILL = -1e18


def _netvlad_kernel(length_ref, x_ref, taps_ref, shift1_ref, w2_ref,
                    bias2_ref, cent_ref, out_ref):
    n = pl.program_id(0)
    T, C = x_ref.shape[1], x_ref.shape[2]
    K = cent_ref.shape[0]

    x = x_ref[0]
    ss = jnp.sum(x * x, axis=1, keepdims=True)
    inv = jax.lax.rsqrt(jnp.maximum(ss, EPS_NORM * EPS_NORM))
    xn = x * inv

    zrow = jnp.zeros((1, C), jnp.float32)
    prev = jnp.concatenate([zrow, xn[:-1, :]], axis=0)
    nxt = jnp.concatenate([xn[1:, :], zrow], axis=0)
    h = (prev * taps_ref[0:1, :] + xn * taps_ref[1:2, :]
         + nxt * taps_ref[2:3, :] + shift1_ref[0:1, :])
    h = jnp.maximum(h, 0.0)

    s = jnp.dot(h, w2_ref[...], preferred_element_type=jnp.float32)
    s = jnp.minimum(jnp.maximum(s + bias2_ref[0:1, :], 0.0), 80.0)

    t_idx = jax.lax.broadcasted_iota(jnp.int32, (T, 1), 0)
    valid = t_idx < length_ref[n]
    s = jnp.where(valid, s, 0.0)
    e = jnp.exp(s)
    a = e * (1.0 / jnp.sum(e, axis=1, keepdims=True))

    vlad = jax.lax.dot_general(a, xn, (((0,), (0,)), ((), ())),
                               preferred_element_type=jnp.float32)
    ones = jnp.ones((T, 1), jnp.float32)
    asum = jax.lax.dot_general(a, ones, (((0,), (0,)), ((), ())),
                               preferred_element_type=jnp.float32)
    vlad = vlad - asum * cent_ref[...]

    n2 = jnp.sum(vlad * vlad, axis=1, keepdims=True)
    vlad = vlad * jax.lax.rsqrt(jnp.maximum(n2, EPS_NORM * EPS_NORM))
    g = jnp.sum(vlad * vlad)
    vlad = vlad * jax.lax.rsqrt(jnp.maximum(g, EPS_NORM * EPS_NORM))
    out_ref[0] = vlad


def kernel(x_, conv1_w, bn1_gamma, bn1_beta, bn1_mean, bn1_var,
           conv2_w, conv2_b, bn2_gamma, bn2_beta, bn2_mean, bn2_var,
           centroids, length):
    N, T, C = x_.shape
    K = centroids.shape[0]

    scale1 = bn1_gamma * jax.lax.rsqrt(bn1_var + EPS_BN)
    shift1 = (bn1_beta - bn1_mean * scale1).reshape(1, C)
    taps = conv1_w[:, 0, :, 1].T * scale1[None, :]

    scale2 = bn2_gamma * jax.lax.rsqrt(bn2_var + EPS_BN)
    w2 = conv2_w[:, :, 0, 0].T * scale2[None, :]
    bias2 = (conv2_b * scale2 + bn2_beta - bn2_mean * scale2).reshape(1, K)

    out = pl.pallas_call(
        _netvlad_kernel,
        grid=(N,),
        in_specs=[
            pl.BlockSpec(memory_space=pltpu.SMEM),
            pl.BlockSpec((1, T, C), lambda n: (n, 0, 0)),
            pl.BlockSpec((3, C), lambda n: (0, 0)),
            pl.BlockSpec((1, C), lambda n: (0, 0)),
            pl.BlockSpec((C, K), lambda n: (0, 0)),
            pl.BlockSpec((1, K), lambda n: (0, 0)),
            pl.BlockSpec((K, C), lambda n: (0, 0)),
        ],
        out_specs=pl.BlockSpec((1, K, C), lambda n: (n, 0, 0)),
        out_shape=jax.ShapeDtypeStruct((N, K, C), jnp.float32),
        compiler_params=pltpu.CompilerParams(
            dimension_semantics=("arbitrary",),
        ),
    )(length, x_, taps, shift1, w2, bias2, centroids)
    return out.reshape(N, K * C)


# speedup vs baseline: 2.9067x; 1.2227x over previous
import jax
import jax.numpy as jnp
from jax.experimental import pallas as pl
from jax.experimental.pallas import tpu as pltpu

EPS_BN = 1e-5
EPS_NORM = 1e-12
NEG_FILL = -1e18


def _netvlad_kernel(length_ref, x_ref, taps_ref, shift1_ref, w2_ref,
                    bias2_ref, cent_ref, out_ref):
    n = pl.program_id(0)
    T, C = x_ref.shape[1], x_ref.shape[2]
    K = cent_ref.shape[0]

    x = x_ref[0]
    ss = jnp.sum(x * x, axis=1, keepdims=True)
    inv = jax.lax.rsqrt(jnp.maximum(ss, EPS_NORM * EPS_NORM))
    xn = x * inv

    zrow = jnp.zeros((1, C), jnp.float32)
    prev = jnp.concatenate([zrow, xn[:-1, :]], axis=0)
    nxt = jnp.concatenate([xn[1:, :], zrow], axis=0)
    h = (prev * taps_ref[0:1, :] + xn * taps_ref[1:2, :]
         + nxt * taps_ref[2:3, :] + shift1_ref[0:1, :])
    h = jnp.maximum(h, 0.0)

    s = jax.lax.dot_general(w2_ref[...], h, (((1,), (1,)), ((), ())),
                            preferred_element_type=jnp.float32)
    s = jnp.minimum(jnp.maximum(s + bias2_ref[...], 0.0), 80.0)

    t_idx = jax.lax.broadcasted_iota(jnp.int32, (1, T), 1)
    valid = t_idx < length_ref[n]
    s = jnp.where(valid, s, 0.0)
    e = jnp.exp(s)
    a = e * (1.0 / jnp.sum(e, axis=0, keepdims=True))

    vlad = jnp.dot(a, xn, preferred_element_type=jnp.float32)
    ones = jnp.ones((T, 1), jnp.float32)
    asum = jnp.dot(a, ones, preferred_element_type=jnp.float32)
    vlad = vlad - asum * cent_ref[...]

    n2 = jnp.sum(vlad * vlad, axis=1, keepdims=True)
    vlad = vlad * jax.lax.rsqrt(jnp.maximum(n2, EPS_NORM * EPS_NORM))
    g = jnp.sum(vlad * vlad)
    vlad = vlad * jax.lax.rsqrt(jnp.maximum(g, EPS_NORM * EPS_NORM))
    out_ref[0] = vlad


def kernel(x_, conv1_w, bn1_gamma, bn1_beta, bn1_mean, bn1_var,
           conv2_w, conv2_b, bn2_gamma, bn2_beta, bn2_mean, bn2_var,
           centroids, length):
    N, T, C = x_.shape
    K = centroids.shape[0]

    scale1 = bn1_gamma * jax.lax.rsqrt(bn1_var + EPS_BN)
    shift1 = (bn1_beta - bn1_mean * scale1).reshape(1, C)
    taps = conv1_w[:, 0, :, 1].T * scale1[None, :]

    scale2 = bn2_gamma * jax.lax.rsqrt(bn2_var + EPS_BN)
    w2 = conv2_w[:, :, 0, 0] * scale2[:, None]
    bias2 = (conv2_b * scale2 + bn2_beta - bn2_mean * scale2).reshape(K, 1)

    out = pl.pallas_call(
        _netvlad_kernel,
        grid=(N,),
        in_specs=[
            pl.BlockSpec(memory_space=pltpu.SMEM),
            pl.BlockSpec((1, T, C), lambda n: (n, 0, 0)),
            pl.BlockSpec((3, C), lambda n: (0, 0)),
            pl.BlockSpec((1, C), lambda n: (0, 0)),
            pl.BlockSpec((K, C), lambda n: (0, 0)),
            pl.BlockSpec((K, 1), lambda n: (0, 0)),
            pl.BlockSpec((K, C), lambda n: (0, 0)),
        ],
        out_specs=pl.BlockSpec((1, K, C), lambda n: (n, 0, 0)),
        out_shape=jax.ShapeDtypeStruct((N, K, C), jnp.float32),
        compiler_params=pltpu.CompilerParams(
            dimension_semantics=("arbitrary",),
        ),
    )(length, x_, taps, shift1, w2, bias2, centroids)
    return out.reshape(N, K * C)
